# baseline (speedup 1.0000x reference)
import functools

import jax
import jax.numpy as jnp
from jax import lax
from jax.experimental import pallas as pl
from jax.experimental.pallas import tpu as pltpu
from jax.experimental.pallas import tpu_sc as plsc

_VOCAB = 30522
_D = 768
_B = 4096
_S = 50
_SP = 56

_NC = 2
_NS = 16
_NW = _NC * _NS
_BW = _B // _NW
_SUB = 2
_CHUNK = _BW // _SUB
_NCHUNKS = _S * _SUB
_NBUF = 2


def _gather_body(idx_hbm, table_hbm, out_hbm, idx_v, *bufs_and_sems):
    rows = bufs_and_sems[:_NBUF]
    gsem = bufs_and_sems[_NBUF:2 * _NBUF]
    ssem = bufs_and_sems[2 * _NBUF:3 * _NBUF]

    wid = lax.axis_index("s") * _NC + lax.axis_index("c")
    base_b = wid * _BW
    pltpu.sync_copy(idx_hbm.at[:, pl.ds(base_b, _BW)], idx_v)

    def idx_slice(i):
        s = i // _SUB
        sub = i % _SUB
        return idx_v.at[s, pl.ds(sub * _CHUNK, _CHUNK)]

    def out_slice(i):
        s = i // _SUB
        sub = i % _SUB
        return out_hbm.at[s, pl.ds(base_b + sub * _CHUNK, _CHUNK)]

    def start_gather(i, b):
        pltpu.async_copy(table_hbm.at[idx_slice(i)], rows[b], gsem[b])

    def wait_gather(i, b):
        pltpu.make_async_copy(table_hbm.at[idx_slice(i)], rows[b], gsem[b]).wait()

    def start_store(i, b):
        pltpu.async_copy(rows[b], out_slice(i), ssem[b])

    def wait_store(i, b):
        pltpu.make_async_copy(rows[b], out_slice(i), ssem[b]).wait()

    for b in range(_NBUF):
        start_gather(b, b)

    wait_gather(0, 0)
    start_store(0, 0)

    def quad(q, carry):
        for r in range(_NBUF):
            i = _NBUF * q + r + 1
            b_prev = r
            b_cur = (r + 1) % _NBUF
            wait_store(i - 1, b_prev)
            start_gather(i + _NBUF - 1, b_prev)
            wait_gather(i, b_cur)
            start_store(i, b_cur)
        return carry

    _NMAIN = (_NCHUNKS - _NBUF) // _NBUF
    lax.fori_loop(0, _NMAIN, quad, 0)

    for i in range(_NCHUNKS - _NBUF + 1, _NCHUNKS):
        b_prev = (i - 1) % _NBUF
        b_cur = i % _NBUF
        wait_store(i - 1, b_prev)
        wait_gather(i, b_cur)
        start_store(i, b_cur)
    wait_store(_NCHUNKS - 1, (_NCHUNKS - 1) % _NBUF)


@jax.jit
def _gather(idx_t, table):
    mesh = plsc.VectorSubcoreMesh(core_axis_name="c", subcore_axis_name="s")
    f = functools.partial(
        pl.kernel,
        out_type=jax.ShapeDtypeStruct((_S, _B, _D), jnp.float32),
        mesh=mesh,
        scratch_types=[pltpu.VMEM((_SP, _BW), jnp.int32)]
        + [pltpu.VMEM((_CHUNK, _D), jnp.float32) for _ in range(_NBUF)]
        + [pltpu.SemaphoreType.DMA for _ in range(2 * _NBUF)],
    )(_gather_body)
    return f(idx_t, table)


def kernel(inputs, table):
    idx_t = jnp.pad(inputs.T.astype(jnp.int32), ((0, _SP - _S), (0, 0)))
    out = _gather(idx_t, table)
    return jnp.transpose(out, (1, 0, 2))


# speedup vs baseline: 1.0127x; 1.0127x over previous
import functools

import jax
import jax.numpy as jnp
from jax import lax
from jax.experimental import pallas as pl
from jax.experimental.pallas import tpu as pltpu
from jax.experimental.pallas import tpu_sc as plsc

_VOCAB = 30522
_D = 768
_B = 4096
_S = 50
_SP = 56

_NC = 2
_NS = 16
_NW = _NC * _NS
_BW = _B // _NW
_SUB = 8
_CHUNK = _BW // _SUB
_NCHUNKS = _S * _SUB
_NBUF = 8


def _gather_body(idx_hbm, table_hbm, out_hbm, idx_v, *bufs_and_sems):
    rows = bufs_and_sems[:_NBUF]
    gsem = bufs_and_sems[_NBUF:2 * _NBUF]
    ssem = bufs_and_sems[2 * _NBUF:3 * _NBUF]

    wid = lax.axis_index("s") * _NC + lax.axis_index("c")
    base_b = wid * _BW
    pltpu.sync_copy(idx_hbm.at[:, pl.ds(base_b, _BW)], idx_v)

    def idx_slice(i):
        s = i // _SUB
        sub = i % _SUB
        return idx_v.at[s, pl.ds(sub * _CHUNK, _CHUNK)]

    def out_slice(i):
        s = i // _SUB
        sub = i % _SUB
        return out_hbm.at[s, pl.ds(base_b + sub * _CHUNK, _CHUNK)]

    def start_gather(i, b):
        pltpu.async_copy(table_hbm.at[idx_slice(i)], rows[b], gsem[b])

    def wait_gather(i, b):
        pltpu.make_async_copy(table_hbm.at[idx_slice(i)], rows[b], gsem[b]).wait()

    def start_store(i, b):
        pltpu.async_copy(rows[b], out_slice(i), ssem[b])

    def wait_store(i, b):
        pltpu.make_async_copy(rows[b], out_slice(i), ssem[b]).wait()

    for b in range(_NBUF):
        start_gather(b, b)

    wait_gather(0, 0)
    start_store(0, 0)

    def quad(q, carry):
        for r in range(_NBUF):
            i = _NBUF * q + r + 1
            b_prev = r
            b_cur = (r + 1) % _NBUF
            wait_store(i - 1, b_prev)
            start_gather(i + _NBUF - 1, b_prev)
            wait_gather(i, b_cur)
            start_store(i, b_cur)
        return carry

    _NMAIN = (_NCHUNKS - _NBUF) // _NBUF
    lax.fori_loop(0, _NMAIN, quad, 0)

    for i in range(_NCHUNKS - _NBUF + 1, _NCHUNKS):
        b_prev = (i - 1) % _NBUF
        b_cur = i % _NBUF
        wait_store(i - 1, b_prev)
        wait_gather(i, b_cur)
        start_store(i, b_cur)
    wait_store(_NCHUNKS - 1, (_NCHUNKS - 1) % _NBUF)


@jax.jit
def _gather(idx_t, table):
    mesh = plsc.VectorSubcoreMesh(core_axis_name="c", subcore_axis_name="s")
    f = functools.partial(
        pl.kernel,
        out_type=jax.ShapeDtypeStruct((_S, _B, _D), jnp.float32),
        mesh=mesh,
        scratch_types=[pltpu.VMEM((_SP, _BW), jnp.int32)]
        + [pltpu.VMEM((_CHUNK, _D), jnp.float32) for _ in range(_NBUF)]
        + [pltpu.SemaphoreType.DMA for _ in range(2 * _NBUF)],
    )(_gather_body)
    return f(idx_t, table)


def kernel(inputs, table):
    idx_t = jnp.pad(inputs.T.astype(jnp.int32), ((0, _SP - _S), (0, 0)))
    out = _gather(idx_t, table)
    return jnp.transpose(out, (1, 0, 2))
